# baseline (speedup 1.0000x reference)
import functools

import jax
import jax.numpy as jnp
from jax.experimental import pallas as pl
from jax.experimental.pallas import tpu as pltpu


def _tree(vals, op):
    while len(vals) > 1:
        nxt = [op(vals[k], vals[k + 1]) for k in range(0, len(vals) - 1, 2)]
        if len(vals) % 2:
            nxt.append(vals[-1])
        vals = nxt
    return vals[0]


def _half_partial(x_ref, l_ref, *, l_row0, eps, ignore_lb, wf):
    _, C, th, W = x_ref.shape
    ql = None
    qc = None
    for g in range(th // 8):
        rows = pl.ds(g * 8, 8)
        lbl = l_ref[0, pl.ds(l_row0 + g * 8, 8), :]
        xs = [x_ref[0, c, rows, :] for c in range(C)]
        m = _tree(xs, jnp.maximum)
        es = [jnp.exp(xc - m) for xc in xs]
        s = _tree(list(es), jnp.add)
        inv = 1.0 / s
        terms = []
        for c in range(C):
            p = es[c] * inv
            d = p - jnp.where(lbl == c, 1.0, 0.0)
            terms.append((eps + 1.0) - d * d)
        prod = _tree(terms, jnp.multiply)
        loss = -jnp.log(prod)
        valid = lbl != ignore_lb
        loss = jnp.where(valid, loss, 0.0)
        cnt = valid.astype(jnp.float32)
        ql = loss if ql is None else ql + loss
        qc = cnt if qc is None else qc + cnt

    out = []
    for arr in (ql, qc):
        q = arr[:, 0:wf]
        for k in range(1, W // wf):
            q = q + arr[:, k * wf:(k + 1) * wf]
        out.append(q)
    return out


def _dfl_block_kernel(*refs, eps, ignore_lb, wf, streams):
    x_refs = refs[:streams]
    l_ref = refs[streams]
    out_ref = refs[streams + 1]
    acc_ref = refs[streams + 2]
    i = pl.program_id(0)
    num_i = pl.num_programs(0)

    @pl.when(i == 0)
    def _():
        acc_ref[...] = jnp.zeros_like(acc_ref)

    th = x_refs[0].shape[2]
    for k, x_ref in enumerate(x_refs):
        ql, qc = _half_partial(x_ref, l_ref, l_row0=k * th,
                               eps=eps, ignore_lb=ignore_lb, wf=wf)
        acc_ref[0] += ql
        acc_ref[1] += qc

    @pl.when(i == num_i - 1)
    def _():
        a = acc_ref[...]
        ls = jnp.sum(a[0], axis=0, keepdims=True)
        nv = jnp.sum(a[1], axis=0, keepdims=True)
        ls = jnp.sum(ls, axis=1, keepdims=True)
        nv = jnp.sum(nv, axis=1, keepdims=True)
        out_ref[...] = ls / nv


def _dual_focal_loss_mean(logits, label, *, ignore_lb, eps, tile_h, streams):
    N, C, H, W = logits.shape

    S = streams
    while H % (S * 8) and S > 1:
        S //= 2
    th = max(8, (tile_h // 8) * 8)
    while H % (S * th):
        th -= 8
        if th < 8:
            th = H // S
            break
    splits = H // (S * th)
    G = N * splits

    lbl = label.astype(jnp.int32)
    wf = 128 if W % 128 == 0 else W

    def x_map(k):
        return lambda i: (i // splits, 0, S * (i % splits) + k, 0)

    in_specs = (
        [pl.BlockSpec((1, C, th, W), x_map(k)) for k in range(S)] +
        [pl.BlockSpec((1, S * th, W), lambda i: (i // splits, i % splits, 0))])

    out = pl.pallas_call(
        functools.partial(_dfl_block_kernel, eps=eps, ignore_lb=ignore_lb,
                          wf=wf, streams=S),
        out_shape=jax.ShapeDtypeStruct((1, 1), jnp.float32),
        grid_spec=pltpu.PrefetchScalarGridSpec(
            num_scalar_prefetch=0,
            grid=(G,),
            in_specs=in_specs,
            out_specs=pl.BlockSpec((1, 1), lambda i: (0, 0)),
            scratch_shapes=[pltpu.VMEM((2, 8, wf), jnp.float32)],
        ),
        compiler_params=pltpu.CompilerParams(
            dimension_semantics=("arbitrary",)),
    )(*([logits] * S + [lbl]))
    return out[0, 0]


def kernel(logits, label):
    return _dual_focal_loss_mean(logits, label, ignore_lb=255, eps=1e-5,
                                 tile_h=128, streams=2)


# speedup vs baseline: 1.0182x; 1.0182x over previous
import functools

import jax
import jax.numpy as jnp
from jax.experimental import pallas as pl
from jax.experimental.pallas import tpu as pltpu


def _tree(vals, op):
    while len(vals) > 1:
        nxt = [op(vals[k], vals[k + 1]) for k in range(0, len(vals) - 1, 2)]
        if len(vals) % 2:
            nxt.append(vals[-1])
        vals = nxt
    return vals[0]


def _half_partial(x_ref, l_ref, *, l_row0, eps, ignore_lb, wf):
    _, C, th, W = x_ref.shape
    ql = None
    qc = None
    for g in range(th // 8):
        rows = pl.ds(g * 8, 8)
        lbl = l_ref[0, pl.ds(l_row0 + g * 8, 8), :]
        xs = [x_ref[0, c, rows, :] for c in range(C)]
        m = _tree(xs, jnp.maximum)
        es = [jnp.exp(xc - m) for xc in xs]
        s = _tree(list(es), jnp.add)
        inv = 1.0 / s
        terms = []
        for c in range(C):
            p = es[c] * inv
            d = p - jnp.where(lbl == c, 1.0, 0.0)
            terms.append((eps + 1.0) - d * d)
        prod = _tree(terms, jnp.multiply)
        loss = -jnp.log(prod)
        valid = lbl != ignore_lb
        loss = jnp.where(valid, loss, 0.0)
        cnt = valid.astype(jnp.float32)
        ql = loss if ql is None else ql + loss
        qc = cnt if qc is None else qc + cnt

    out = []
    for arr in (ql, qc):
        q = arr[:, 0:wf]
        for k in range(1, W // wf):
            q = q + arr[:, k * wf:(k + 1) * wf]
        out.append(q)
    return out


def _dfl_block_kernel(*refs, eps, ignore_lb, wf, streams):
    x_refs = refs[:streams]
    l_ref = refs[streams]
    out_ref = refs[streams + 1]
    acc_ref = refs[streams + 2]
    i = pl.program_id(0)
    num_i = pl.num_programs(0)

    @pl.when(i == 0)
    def _():
        acc_ref[...] = jnp.zeros_like(acc_ref)

    th = x_refs[0].shape[2]
    for k, x_ref in enumerate(x_refs):
        ql, qc = _half_partial(x_ref, l_ref, l_row0=k * th,
                               eps=eps, ignore_lb=ignore_lb, wf=wf)
        acc_ref[0] += ql
        acc_ref[1] += qc

    @pl.when(i == num_i - 1)
    def _():
        a = acc_ref[...]
        ls = jnp.sum(a[0], axis=0, keepdims=True)
        nv = jnp.sum(a[1], axis=0, keepdims=True)
        ls = jnp.sum(ls, axis=1, keepdims=True)
        nv = jnp.sum(nv, axis=1, keepdims=True)
        out_ref[...] = ls / nv


def _dual_focal_loss_mean(logits, label, *, ignore_lb, eps, tile_h, streams):
    N, C, H, W = logits.shape

    S = streams
    while H % (S * 8) and S > 1:
        S //= 2
    th = max(8, (tile_h // 8) * 8)
    while H % (S * th):
        th -= 8
        if th < 8:
            th = H // S
            break
    splits = H // (S * th)
    G = N * splits

    lbl = label.astype(jnp.int32)
    wf = 128 if W % 128 == 0 else W

    def x_map(k):
        return lambda i: (i // splits, 0, S * (i % splits) + k, 0)

    in_specs = (
        [pl.BlockSpec((1, C, th, W), x_map(k)) for k in range(S)] +
        [pl.BlockSpec((1, S * th, W), lambda i: (i // splits, i % splits, 0))])

    out = pl.pallas_call(
        functools.partial(_dfl_block_kernel, eps=eps, ignore_lb=ignore_lb,
                          wf=wf, streams=S),
        out_shape=jax.ShapeDtypeStruct((1, 1), jnp.float32),
        grid_spec=pltpu.PrefetchScalarGridSpec(
            num_scalar_prefetch=0,
            grid=(G,),
            in_specs=in_specs,
            out_specs=pl.BlockSpec((1, 1), lambda i: (0, 0)),
            scratch_shapes=[pltpu.VMEM((2, 8, wf), jnp.float32)],
        ),
        compiler_params=pltpu.CompilerParams(
            dimension_semantics=("arbitrary",)),
    )(*([logits] * S + [lbl]))
    return out[0, 0]


def kernel(logits, label):
    return _dual_focal_loss_mean(logits, label, ignore_lb=255, eps=1e-5,
                                 tile_h=256, streams=1)
